# Initial kernel scaffold; baseline (speedup 1.0000x reference)
import jax
import jax.numpy as jnp
from jax.experimental import pallas as pl


def kernel(x, edge_index, disease, W1, as1, ad1, b1, W2, as2, ad2, b2, Ws, bs, Wsig1, bsig1, Wsig2, bsig2, Wrole1, brole1, Wrole2, brole2, Wab1, bab1, Wab2, bab2):
    raise NotImplementedError("write your pallas kernel here")


# speedup vs baseline: 6.6595x; 6.6595x over previous
import functools

import jax
import jax.numpy as jnp
from jax.experimental import pallas as pl
from jax.experimental.pallas import tpu as pltpu

N = 10000
E = 320000
D_IN = 128
HID = 64
ND = 3
SHARED = 128
ROWS = 1000


def _mm(a, b):
    return jax.lax.dot_general(a, b, (((1,), (0,)), ((), ())),
                               preferred_element_type=jnp.float32,
                               precision=jax.lax.Precision.HIGHEST)


def _k_dense1(x_ref, w_ref, asd_ref, h_ref, sd_ref):
    h = _mm(x_ref[...], w_ref[...])
    h_ref[...] = h
    sd_ref[...] = _mm(h, asd_ref[...])


def dense1(x, wcat, asd):
    grid = (N // ROWS,)
    return pl.pallas_call(
        _k_dense1,
        grid=grid,
        in_specs=[
            pl.BlockSpec((ROWS, D_IN), lambda i: (i, 0)),
            pl.BlockSpec((D_IN, ND * HID), lambda i: (0, 0)),
            pl.BlockSpec((ND * HID, 8), lambda i: (0, 0)),
        ],
        out_specs=[
            pl.BlockSpec((ROWS, ND * HID), lambda i: (i, 0)),
            pl.BlockSpec((ROWS, 8), lambda i: (i, 0)),
        ],
        out_shape=[
            jax.ShapeDtypeStruct((N, ND * HID), jnp.float32),
            jax.ShapeDtypeStruct((N, 8), jnp.float32),
        ],
    )(x, wcat, asd)


def _k_dense2(outa_ref, outb_ref, exp_ref, b_ref, w_ref, asd_ref, h_ref, sd_ref):
    acc = outa_ref[...] + outb_ref[...]
    num = acc[:, :192]
    den = acc[:, 192:195]
    den_big = _mm(den, exp_ref[...])
    a = jnp.maximum(num / (den_big + 1e-16) + b_ref[...], 0.0)
    h = _mm(a, w_ref[...])
    h_ref[...] = h
    sd_ref[...] = _mm(h, asd_ref[...])


def dense2(outa, outb, expand, b1, w2bd, asd2):
    grid = (N // ROWS,)
    return pl.pallas_call(
        _k_dense2,
        grid=grid,
        in_specs=[
            pl.BlockSpec((ROWS, 208), lambda i: (i, 0)),
            pl.BlockSpec((ROWS, 208), lambda i: (i, 0)),
            pl.BlockSpec((ND, ND * HID), lambda i: (0, 0)),
            pl.BlockSpec((1, ND * HID), lambda i: (0, 0)),
            pl.BlockSpec((ND * HID, ND * HID), lambda i: (0, 0)),
            pl.BlockSpec((ND * HID, 8), lambda i: (0, 0)),
        ],
        out_specs=[
            pl.BlockSpec((ROWS, ND * HID), lambda i: (i, 0)),
            pl.BlockSpec((ROWS, 8), lambda i: (i, 0)),
        ],
        out_shape=[
            jax.ShapeDtypeStruct((N, ND * HID), jnp.float32),
            jax.ShapeDtypeStruct((N, 8), jnp.float32),
        ],
    )(outa, outb, expand, b1, w2bd, asd2)


def _k_heads(outa_ref, outb_ref, exp_ref, b2_ref, ws_ref, bs_ref,
             wh1_ref, bh1_ref, wh2_ref, bh2_ref, y_ref):
    acc = outa_ref[...] + outb_ref[...]
    num = acc[:, :192]
    den = acc[:, 192:195]
    den_big = _mm(den, exp_ref[...])
    a2 = num / (den_big + 1e-16) + b2_ref[...]
    z = _mm(a2, ws_ref[...]) + bs_ref[...]
    hs = jnp.where(z > 0, z, jnp.exp(jnp.minimum(z, 0.0)) - 1.0)
    t = jnp.maximum(_mm(hs, wh1_ref[...]) + bh1_ref[...], 0.0)
    y_ref[...] = _mm(t, wh2_ref[...]) + bh2_ref[...]


def heads(outa, outb, expand, b2, ws, bs, wh1, bh1, wh2, bh2):
    grid = (N // ROWS,)
    return pl.pallas_call(
        _k_heads,
        grid=grid,
        in_specs=[
            pl.BlockSpec((ROWS, 208), lambda i: (i, 0)),
            pl.BlockSpec((ROWS, 208), lambda i: (i, 0)),
            pl.BlockSpec((ND, ND * HID), lambda i: (0, 0)),
            pl.BlockSpec((1, ND * HID), lambda i: (0, 0)),
            pl.BlockSpec((ND * HID, SHARED), lambda i: (0, 0)),
            pl.BlockSpec((1, SHARED), lambda i: (0, 0)),
            pl.BlockSpec((SHARED, ND * HID), lambda i: (0, 0)),
            pl.BlockSpec((1, ND * HID), lambda i: (0, 0)),
            pl.BlockSpec((ND * HID, 8), lambda i: (0, 0)),
            pl.BlockSpec((1, 8), lambda i: (0, 0)),
        ],
        out_specs=pl.BlockSpec((ROWS, 8), lambda i: (i, 0)),
        out_shape=jax.ShapeDtypeStruct((N, 8), jnp.float32),
    )(outa, outb, expand, b2, ws, bs, wh1, bh1, wh2, bh2)


def edge_phase(h, sd, src, dst):
    s = sd[:, :3]
    d = sd[:, 3:6]
    logit = s[src] + d[dst]
    alpha = jnp.maximum(logit, 0.2 * logit)
    e = jnp.exp(alpha)
    den = jax.ops.segment_sum(e, dst, num_segments=N)
    hw = h[src].reshape(E, ND, HID) * e[:, :, None]
    num = jax.ops.segment_sum(hw.reshape(E, ND * HID), dst, num_segments=N)
    out = jnp.concatenate(
        [num, den, jnp.zeros((N, 13), jnp.float32)], axis=1)
    return out, jnp.zeros_like(out)


def kernel(x, edge_index, disease, W1, as1, ad1, b1, W2, as2, ad2, b2, Ws, bs,
           Wsig1, bsig1, Wsig2, bsig2, Wrole1, brole1, Wrole2, brole2,
           Wab1, bab1, Wab2, bab2):
    src = edge_index[0]
    dst = edge_index[1]

    w1cat = jnp.transpose(W1, (1, 0, 2)).reshape(D_IN, ND * HID)
    eye = jnp.eye(ND, dtype=jnp.float32)
    asd1 = jnp.concatenate([
        jnp.einsum('dh,dc->dhc', as1, eye).reshape(ND * HID, ND),
        jnp.einsum('dh,dc->dhc', ad1, eye).reshape(ND * HID, ND),
        jnp.zeros((ND * HID, 2), jnp.float32)], axis=1)
    asd2 = jnp.concatenate([
        jnp.einsum('dh,dc->dhc', as2, eye).reshape(ND * HID, ND),
        jnp.einsum('dh,dc->dhc', ad2, eye).reshape(ND * HID, ND),
        jnp.zeros((ND * HID, 2), jnp.float32)], axis=1)
    w2bd = jnp.einsum('dij,dc->dicj', W2, eye).reshape(ND * HID, ND * HID)
    expand = jnp.repeat(eye, HID, axis=1)
    b1f = b1.reshape(1, ND * HID)
    b2f = b2.reshape(1, ND * HID)
    wh1 = jnp.concatenate([Wsig1[disease], Wrole1[disease], Wab1[disease]],
                          axis=1)
    bh1 = jnp.concatenate([bsig1[disease], brole1[disease], bab1[disease]],
                          axis=0).reshape(1, ND * HID)
    z64 = jnp.zeros((HID, 1), jnp.float32)
    wh2 = jnp.concatenate([
        jnp.concatenate([Wsig2[disease], z64, z64, z64], axis=1),
        jnp.concatenate([z64, Wrole2[disease], z64], axis=1),
        jnp.concatenate([z64, z64, z64, Wab2[disease]], axis=1),
    ], axis=0)
    wh2 = jnp.concatenate([wh2, jnp.zeros((ND * HID, 4), jnp.float32)], axis=1)
    bh2 = jnp.concatenate([bsig2[disease], brole2[disease], bab2[disease],
                           jnp.zeros((4,), jnp.float32)]).reshape(1, 8)

    h1, sd1 = dense1(x, w1cat, asd1)
    outa, outb = edge_phase(h1, sd1, src, dst)
    h2, sd2 = dense2(outa, outb, expand, b1f, w2bd, asd2)
    outa2, outb2 = edge_phase(h2, sd2, src, dst)
    y = heads(outa2, outb2, expand, b2f, Ws, bs.reshape(1, SHARED),
              wh1, bh1, wh2, bh2)
    return (y[:, 0:1], y[:, 1:3], y[:, 3:4])


# speedup vs baseline: 32.4385x; 4.8710x over previous
import jax
import jax.numpy as jnp
from jax import lax
from jax.experimental import pallas as pl
from jax.experimental.pallas import tpu as pltpu
from jax.experimental.pallas import tpu_sc as plsc

N = 10000
E = 320000
D_IN = 128
HID = 64
ND = 3
SHARED = 128
F = ND * HID
FH = F // 2
ROWS = 1000


def _mm(a, b):
    return jax.lax.dot_general(a, b, (((1,), (0,)), ((), ())),
                               preferred_element_type=jnp.float32)


def _mm_hi(a, b):
    return jax.lax.dot_general(a, b, (((1,), (0,)), ((), ())),
                               preferred_element_type=jnp.float32,
                               precision=jax.lax.Precision.HIGHEST)


def _k_dense1(x_ref, wlo_ref, whi_ref, alo_ref, ahi_ref, h_ref, sd_ref):
    x = x_ref[...]
    hlo = _mm(x, wlo_ref[...])
    hhi = _mm(x, whi_ref[...])
    h_ref[0] = hlo
    h_ref[1] = hhi
    sd_ref[...] = _mm_hi(hlo, alo_ref[...]) + _mm_hi(hhi, ahi_ref[...])


def dense1(x, wlo, whi, asdlo, asdhi):
    return pl.pallas_call(
        _k_dense1,
        grid=(N // ROWS,),
        in_specs=[
            pl.BlockSpec((ROWS, D_IN), lambda i: (i, 0)),
            pl.BlockSpec((D_IN, FH), lambda i: (0, 0)),
            pl.BlockSpec((D_IN, FH), lambda i: (0, 0)),
            pl.BlockSpec((FH, 16), lambda i: (0, 0)),
            pl.BlockSpec((FH, 16), lambda i: (0, 0)),
        ],
        out_specs=[
            pl.BlockSpec((2, ROWS, FH), lambda i: (0, i, 0)),
            pl.BlockSpec((ROWS, 16), lambda i: (i, 0)),
        ],
        out_shape=[
            jax.ShapeDtypeStruct((2, N, FH), jnp.float32),
            jax.ShapeDtypeStruct((N, 16), jnp.float32),
        ],
    )(x, wlo, whi, asdlo, asdhi)


def _norm_act(out_ref, exp_ref, b_ref, relu):
    num = out_ref[...][:, :FH]
    den = out_ref[...][:, FH:FH + 3]
    den_big = _mm_hi(den, exp_ref[...])
    a = num / (den_big + 1e-16) + b_ref[...]
    return jnp.maximum(a, 0.0) if relu else a


def _k_dense2(outl_ref, outh_ref, explo_ref, exphi_ref, blo_ref, bhi_ref,
              wll_ref, whl_ref, wlh_ref, whh_ref, alo_ref, ahi_ref,
              h_ref, sd_ref):
    alo = _norm_act(outl_ref, explo_ref, blo_ref, True)
    ahi = _norm_act(outh_ref, exphi_ref, bhi_ref, True)
    hlo = _mm(alo, wll_ref[...]) + _mm(ahi, whl_ref[...])
    hhi = _mm(alo, wlh_ref[...]) + _mm(ahi, whh_ref[...])
    h_ref[0] = hlo
    h_ref[1] = hhi
    sd_ref[...] = _mm_hi(hlo, alo_ref[...]) + _mm_hi(hhi, ahi_ref[...])


def dense2(outl, outh, explo, exphi, blo, bhi, wll, whl, wlh, whh,
           asdlo, asdhi):
    return pl.pallas_call(
        _k_dense2,
        grid=(N // ROWS,),
        in_specs=[
            pl.BlockSpec((ROWS, 112), lambda i: (i, 0)),
            pl.BlockSpec((ROWS, 112), lambda i: (i, 0)),
            pl.BlockSpec((ND, FH), lambda i: (0, 0)),
            pl.BlockSpec((ND, FH), lambda i: (0, 0)),
            pl.BlockSpec((1, FH), lambda i: (0, 0)),
            pl.BlockSpec((1, FH), lambda i: (0, 0)),
            pl.BlockSpec((FH, FH), lambda i: (0, 0)),
            pl.BlockSpec((FH, FH), lambda i: (0, 0)),
            pl.BlockSpec((FH, FH), lambda i: (0, 0)),
            pl.BlockSpec((FH, FH), lambda i: (0, 0)),
            pl.BlockSpec((FH, 16), lambda i: (0, 0)),
            pl.BlockSpec((FH, 16), lambda i: (0, 0)),
        ],
        out_specs=[
            pl.BlockSpec((2, ROWS, FH), lambda i: (0, i, 0)),
            pl.BlockSpec((ROWS, 16), lambda i: (i, 0)),
        ],
        out_shape=[
            jax.ShapeDtypeStruct((2, N, FH), jnp.float32),
            jax.ShapeDtypeStruct((N, 16), jnp.float32),
        ],
    )(outl, outh, explo, exphi, blo, bhi, wll, whl, wlh, whh, asdlo, asdhi)


def _k_heads(outl_ref, outh_ref, explo_ref, exphi_ref, blo_ref, bhi_ref,
             wslo_ref, wshi_ref, bs_ref, wh1_ref, bh1_ref, wh2_ref, bh2_ref,
             y_ref):
    alo = _norm_act(outl_ref, explo_ref, blo_ref, False)
    ahi = _norm_act(outh_ref, exphi_ref, bhi_ref, False)
    z = _mm(alo, wslo_ref[...]) + _mm(ahi, wshi_ref[...]) + bs_ref[...]
    hs = jnp.where(z > 0, z, jnp.exp(jnp.minimum(z, 0.0)) - 1.0)
    t = jnp.maximum(_mm(hs, wh1_ref[...]) + bh1_ref[...], 0.0)
    y_ref[...] = _mm(t, wh2_ref[...]) + bh2_ref[...]


def heads(outl, outh, explo, exphi, blo, bhi, wslo, wshi, bs, wh1, bh1,
          wh2, bh2):
    return pl.pallas_call(
        _k_heads,
        grid=(N // ROWS,),
        in_specs=[
            pl.BlockSpec((ROWS, 112), lambda i: (i, 0)),
            pl.BlockSpec((ROWS, 112), lambda i: (i, 0)),
            pl.BlockSpec((ND, FH), lambda i: (0, 0)),
            pl.BlockSpec((ND, FH), lambda i: (0, 0)),
            pl.BlockSpec((1, FH), lambda i: (0, 0)),
            pl.BlockSpec((1, FH), lambda i: (0, 0)),
            pl.BlockSpec((FH, SHARED), lambda i: (0, 0)),
            pl.BlockSpec((FH, SHARED), lambda i: (0, 0)),
            pl.BlockSpec((1, SHARED), lambda i: (0, 0)),
            pl.BlockSpec((SHARED, F), lambda i: (0, 0)),
            pl.BlockSpec((1, F), lambda i: (0, 0)),
            pl.BlockSpec((F, 8), lambda i: (0, 0)),
            pl.BlockSpec((1, 8), lambda i: (0, 0)),
        ],
        out_specs=pl.BlockSpec((ROWS, 8), lambda i: (i, 0)),
        out_shape=jax.ShapeDtypeStruct((N, 8), jnp.float32),
    )(outl, outh, explo, exphi, blo, bhi, wslo, wshi, bs, wh1, bh1, wh2, bh2)


EK = 80
TPC = 16
EPT = E // TPC
NCH = EPT // EK
OUTW = 112
ZROWS = 1000


def _edge_body(h_hbm, sd_hbm, src_hbm, dst_hbm, out_hbm,
               srcb, dstb, sdsrc, sddst, hrows, scaled, eb, acc,
               sem1, sem2, sem3):
    c = lax.axis_index("c")
    sid = lax.axis_index("s")
    base = sid * EPT

    def _zero_row(r, carry):
        for q in range(OUTW // 16):
            scaled[r, pl.ds(q * 16, 16)] = jnp.zeros((16,), jnp.float32)
        return carry
    lax.fori_loop(0, EK, _zero_row, 0)

    @pl.when(sid < 10)
    def _():
        for q in range(ZROWS // EK):
            pltpu.sync_copy(scaled, acc.at[pl.ds(sid * ZROWS + q * EK, EK)])
        rem = ZROWS - (ZROWS // EK) * EK
        if rem:
            pltpu.sync_copy(scaled.at[pl.ds(0, rem)],
                            acc.at[pl.ds(sid * ZROWS + (ZROWS // EK) * EK, rem)])
    plsc.subcore_barrier()

    lane = lax.broadcasted_iota(jnp.int32, (16,), 0)

    def _chunk(ch, carry):
        off = base + ch * EK
        pltpu.sync_copy(src_hbm.at[pl.ds(off, EK)], srcb)
        pltpu.sync_copy(dst_hbm.at[pl.ds(off, EK)], dstb)
        cp1 = pltpu.async_copy(sd_hbm.at[srcb], sdsrc, sem1)
        cp2 = pltpu.async_copy(sd_hbm.at[dstb], sddst, sem2)
        cp3 = pltpu.async_copy(h_hbm.at[c].at[srcb], hrows, sem3)
        cp1.wait()
        cp2.wait()
        cp3.wait()
        for jj in range(EK // 16):
            rows = lane + jj * 16
            for dd in range(ND):
                a = (plsc.load_gather(sdsrc, [rows, jnp.full((16,), dd, jnp.int32)])
                     + plsc.load_gather(sddst, [rows, jnp.full((16,), dd + 3, jnp.int32)]))
                eb[dd, pl.ds(jj * 16, 16)] = jnp.exp(jnp.maximum(a, 0.2 * a))

        @plsc.parallel_loop(0, EK)
        def _scale(j):
            jsel = jnp.full((16,), j, jnp.int32)
            ev = [plsc.load_gather(eb, [jnp.full((16,), dd, jnp.int32), jsel])
                  for dd in range(ND)]
            for q in range(FH // 16):
                dq = (c * FH + q * 16) // HID
                evq = jnp.where(dq == 0, ev[0],
                                jnp.where(dq == 1, ev[1], ev[2]))
                scaled[j, pl.ds(q * 16, 16)] = hrows[j, pl.ds(q * 16, 16)] * evq
            denv = jnp.where(lane == 0, ev[0],
                             jnp.where(lane == 1, ev[1],
                                       jnp.where(lane == 2, ev[2], 0.0)))
            scaled[j, pl.ds(FH, 16)] = denv

        pltpu.sync_copy(scaled, acc.at[dstb], add=True)
        return carry

    lax.fori_loop(0, NCH, _chunk, 0)

    plsc.subcore_barrier()
    @pl.when(sid < 10)
    def _():
        pltpu.sync_copy(acc.at[pl.ds(sid * ZROWS, ZROWS)],
                        out_hbm.at[c].at[pl.ds(sid * ZROWS, ZROWS)])


def edge_conv(hsplit, sd, src, dst):
    mesh = plsc.VectorSubcoreMesh(core_axis_name="c", subcore_axis_name="s")
    f = pl.kernel(
        _edge_body,
        out_type=jax.ShapeDtypeStruct((2, N, OUTW), jnp.float32),
        mesh=mesh,
        compiler_params=pltpu.CompilerParams(use_tc_tiling_on_sc=False,
                                             needs_layout_passes=False),
        scratch_types=[
            pltpu.VMEM((EK,), jnp.int32),
            pltpu.VMEM((EK,), jnp.int32),
            pltpu.VMEM((EK, 16), jnp.float32),
            pltpu.VMEM((EK, 16), jnp.float32),
            pltpu.VMEM((EK, FH), jnp.float32),
            pltpu.VMEM((EK, OUTW), jnp.float32),
            pltpu.VMEM((ND, EK), jnp.float32),
            pltpu.VMEM_SHARED((N, OUTW), jnp.float32),
            pltpu.SemaphoreType.DMA,
            pltpu.SemaphoreType.DMA,
            pltpu.SemaphoreType.DMA,
        ],
    )
    return f(hsplit, sd, src, dst)


def kernel(x, edge_index, disease, W1, as1, ad1, b1, W2, as2, ad2, b2, Ws, bs,
           Wsig1, bsig1, Wsig2, bsig2, Wrole1, brole1, Wrole2, brole2,
           Wab1, bab1, Wab2, bab2):
    src = edge_index[0]
    dst = edge_index[1]

    eye = jnp.eye(ND, dtype=jnp.float32)
    w1cat = jnp.transpose(W1, (1, 0, 2)).reshape(D_IN, F)
    def asd_pack(a_s, a_d):
        m = jnp.concatenate([
            jnp.einsum('dh,dc->dhc', a_s, eye).reshape(F, ND),
            jnp.einsum('dh,dc->dhc', a_d, eye).reshape(F, ND),
            jnp.zeros((F, 10), jnp.float32)], axis=1)
        return m[:FH], m[FH:]
    asd1lo, asd1hi = asd_pack(as1, ad1)
    asd2lo, asd2hi = asd_pack(as2, ad2)
    w2bd = jnp.einsum('dij,dc->dicj', W2, eye).reshape(F, F)
    expand = jnp.repeat(eye, HID, axis=1)
    explo, exphi = expand[:, :FH], expand[:, FH:]
    b1f = b1.reshape(1, F)
    b2f = b2.reshape(1, F)
    wh1 = jnp.concatenate([Wsig1[disease], Wrole1[disease], Wab1[disease]],
                          axis=1)
    bh1 = jnp.concatenate([bsig1[disease], brole1[disease], bab1[disease]],
                          axis=0).reshape(1, F)
    z64 = jnp.zeros((HID, 1), jnp.float32)
    wh2 = jnp.concatenate([
        jnp.concatenate([Wsig2[disease], z64, z64, z64], axis=1),
        jnp.concatenate([z64, Wrole2[disease], z64], axis=1),
        jnp.concatenate([z64, z64, z64, Wab2[disease]], axis=1),
    ], axis=0)
    wh2 = jnp.concatenate([wh2, jnp.zeros((F, 4), jnp.float32)], axis=1)
    bh2 = jnp.concatenate([bsig2[disease], brole2[disease], bab2[disease],
                           jnp.zeros((4,), jnp.float32)]).reshape(1, 8)

    h1, sd1 = dense1(x, w1cat[:, :FH], w1cat[:, FH:], asd1lo, asd1hi)
    o1 = edge_conv(h1, sd1, src, dst)
    h2, sd2 = dense2(o1[0], o1[1], explo, exphi, b1f[:, :FH], b1f[:, FH:],
                     w2bd[:FH, :FH], w2bd[FH:, :FH], w2bd[:FH, FH:],
                     w2bd[FH:, FH:], asd2lo, asd2hi)
    o2 = edge_conv(h2, sd2, src, dst)
    y = heads(o2[0], o2[1], explo, exphi, b2f[:, :FH], b2f[:, FH:],
              Ws[:FH], Ws[FH:], bs.reshape(1, SHARED), wh1, bh1, wh2, bh2)
    return (y[:, 0:1], y[:, 1:3], y[:, 3:4])


# speedup vs baseline: 65.5888x; 2.0219x over previous
import jax
import jax.numpy as jnp
from jax import lax
from jax.experimental import pallas as pl
from jax.experimental.pallas import tpu as pltpu
from jax.experimental.pallas import tpu_sc as plsc

N = 10000
E = 320000
D_IN = 128
HID = 64
ND = 3
SHARED = 128
F = ND * HID
FH = F // 2
ROWS = 1000


def _mm(a, b):
    return jax.lax.dot_general(a, b, (((1,), (0,)), ((), ())),
                               preferred_element_type=jnp.float32)


def _mm_hi(a, b):
    return jax.lax.dot_general(a, b, (((1,), (0,)), ((), ())),
                               preferred_element_type=jnp.float32,
                               precision=jax.lax.Precision.HIGHEST)


def _k_dense1(x_ref, wlo_ref, whi_ref, alo_ref, ahi_ref, h_ref, sd_ref):
    x = x_ref[...]
    hlo = _mm(x, wlo_ref[...])
    hhi = _mm(x, whi_ref[...])
    h_ref[0] = hlo
    h_ref[1] = hhi
    sd_ref[...] = _mm_hi(hlo, alo_ref[...]) + _mm_hi(hhi, ahi_ref[...])


def dense1(x, wlo, whi, asdlo, asdhi):
    return pl.pallas_call(
        _k_dense1,
        grid=(N // ROWS,),
        in_specs=[
            pl.BlockSpec((ROWS, D_IN), lambda i: (i, 0)),
            pl.BlockSpec((D_IN, FH), lambda i: (0, 0)),
            pl.BlockSpec((D_IN, FH), lambda i: (0, 0)),
            pl.BlockSpec((FH, 16), lambda i: (0, 0)),
            pl.BlockSpec((FH, 16), lambda i: (0, 0)),
        ],
        out_specs=[
            pl.BlockSpec((2, ROWS, FH), lambda i: (0, i, 0)),
            pl.BlockSpec((ROWS, 16), lambda i: (i, 0)),
        ],
        out_shape=[
            jax.ShapeDtypeStruct((2, N, FH), jnp.float32),
            jax.ShapeDtypeStruct((N, 16), jnp.float32),
        ],
    )(x, wlo, whi, asdlo, asdhi)


def _norm_act(out_ref, exp_ref, b_ref, relu):
    num = out_ref[...][:, :FH]
    den = out_ref[...][:, FH:FH + 3]
    den_big = _mm_hi(den, exp_ref[...])
    a = num / (den_big + 1e-16) + b_ref[...]
    return jnp.maximum(a, 0.0) if relu else a


def _k_dense2(outl_ref, outh_ref, explo_ref, exphi_ref, blo_ref, bhi_ref,
              wll_ref, whl_ref, wlh_ref, whh_ref, alo_ref, ahi_ref,
              h_ref, sd_ref):
    alo = _norm_act(outl_ref, explo_ref, blo_ref, True)
    ahi = _norm_act(outh_ref, exphi_ref, bhi_ref, True)
    hlo = _mm(alo, wll_ref[...]) + _mm(ahi, whl_ref[...])
    hhi = _mm(alo, wlh_ref[...]) + _mm(ahi, whh_ref[...])
    h_ref[0] = hlo
    h_ref[1] = hhi
    sd_ref[...] = _mm_hi(hlo, alo_ref[...]) + _mm_hi(hhi, ahi_ref[...])


def dense2(outl, outh, explo, exphi, blo, bhi, wll, whl, wlh, whh,
           asdlo, asdhi):
    return pl.pallas_call(
        _k_dense2,
        grid=(N // ROWS,),
        in_specs=[
            pl.BlockSpec((ROWS, 112), lambda i: (i, 0)),
            pl.BlockSpec((ROWS, 112), lambda i: (i, 0)),
            pl.BlockSpec((ND, FH), lambda i: (0, 0)),
            pl.BlockSpec((ND, FH), lambda i: (0, 0)),
            pl.BlockSpec((1, FH), lambda i: (0, 0)),
            pl.BlockSpec((1, FH), lambda i: (0, 0)),
            pl.BlockSpec((FH, FH), lambda i: (0, 0)),
            pl.BlockSpec((FH, FH), lambda i: (0, 0)),
            pl.BlockSpec((FH, FH), lambda i: (0, 0)),
            pl.BlockSpec((FH, FH), lambda i: (0, 0)),
            pl.BlockSpec((FH, 16), lambda i: (0, 0)),
            pl.BlockSpec((FH, 16), lambda i: (0, 0)),
        ],
        out_specs=[
            pl.BlockSpec((2, ROWS, FH), lambda i: (0, i, 0)),
            pl.BlockSpec((ROWS, 16), lambda i: (i, 0)),
        ],
        out_shape=[
            jax.ShapeDtypeStruct((2, N, FH), jnp.float32),
            jax.ShapeDtypeStruct((N, 16), jnp.float32),
        ],
    )(outl, outh, explo, exphi, blo, bhi, wll, whl, wlh, whh, asdlo, asdhi)


def _k_heads(outl_ref, outh_ref, explo_ref, exphi_ref, blo_ref, bhi_ref,
             wslo_ref, wshi_ref, bs_ref, wh1_ref, bh1_ref, wh2_ref, bh2_ref,
             y_ref):
    alo = _norm_act(outl_ref, explo_ref, blo_ref, False)
    ahi = _norm_act(outh_ref, exphi_ref, bhi_ref, False)
    z = _mm(alo, wslo_ref[...]) + _mm(ahi, wshi_ref[...]) + bs_ref[...]
    hs = jnp.where(z > 0, z, jnp.exp(jnp.minimum(z, 0.0)) - 1.0)
    t = jnp.maximum(_mm(hs, wh1_ref[...]) + bh1_ref[...], 0.0)
    y_ref[...] = _mm(t, wh2_ref[...]) + bh2_ref[...]


def heads(outl, outh, explo, exphi, blo, bhi, wslo, wshi, bs, wh1, bh1,
          wh2, bh2):
    return pl.pallas_call(
        _k_heads,
        grid=(N // ROWS,),
        in_specs=[
            pl.BlockSpec((ROWS, 112), lambda i: (i, 0)),
            pl.BlockSpec((ROWS, 112), lambda i: (i, 0)),
            pl.BlockSpec((ND, FH), lambda i: (0, 0)),
            pl.BlockSpec((ND, FH), lambda i: (0, 0)),
            pl.BlockSpec((1, FH), lambda i: (0, 0)),
            pl.BlockSpec((1, FH), lambda i: (0, 0)),
            pl.BlockSpec((FH, SHARED), lambda i: (0, 0)),
            pl.BlockSpec((FH, SHARED), lambda i: (0, 0)),
            pl.BlockSpec((1, SHARED), lambda i: (0, 0)),
            pl.BlockSpec((SHARED, F), lambda i: (0, 0)),
            pl.BlockSpec((1, F), lambda i: (0, 0)),
            pl.BlockSpec((F, 8), lambda i: (0, 0)),
            pl.BlockSpec((1, 8), lambda i: (0, 0)),
        ],
        out_specs=pl.BlockSpec((ROWS, 8), lambda i: (i, 0)),
        out_shape=jax.ShapeDtypeStruct((N, 8), jnp.float32),
    )(outl, outh, explo, exphi, blo, bhi, wslo, wshi, bs, wh1, bh1, wh2, bh2)


EK = 80
TPC = 16
EPT = E // TPC
NCH = EPT // EK
OUTW = 112
ZROWS = 1000


def _edge_body(h_hbm, sd_hbm, src_hbm, dst_hbm, out_hbm,
               srcb, dstb, dsts, sdsrc, sddst, hrows, scaled, eb, acc,
               isem, gsem, ssem):
    c = lax.axis_index("c")
    sid = lax.axis_index("s")
    base = sid * EPT

    def _zero_row(r, carry):
        for q in range(OUTW // 16):
            scaled[0][r, pl.ds(q * 16, 16)] = jnp.zeros((16,), jnp.float32)
        return carry
    lax.fori_loop(0, EK, _zero_row, 0)

    @pl.when(sid < 10)
    def _():
        for q in range(ZROWS // EK):
            pltpu.sync_copy(scaled[0], acc.at[pl.ds(sid * ZROWS + q * EK, EK)])
        rem = ZROWS - (ZROWS // EK) * EK
        if rem:
            pltpu.sync_copy(scaled[0].at[pl.ds(0, rem)],
                            acc.at[pl.ds(sid * ZROWS + (ZROWS // EK) * EK, rem)])
    plsc.subcore_barrier()

    lane = lax.broadcasted_iota(jnp.int32, (16,), 0)

    def _issue_idx(ch, b):
        off = base + ch * EK
        pltpu.async_copy(src_hbm.at[pl.ds(off, EK)], srcb[b], isem[b])
        pltpu.async_copy(dst_hbm.at[pl.ds(off, EK)], dstb[b], isem[b])

    def _drain_idx(ch, b):
        off = base + ch * EK
        pltpu.make_async_copy(src_hbm.at[pl.ds(off, EK)], srcb[b], isem[b]).wait()
        pltpu.make_async_copy(dst_hbm.at[pl.ds(off, EK)], dstb[b], isem[b]).wait()

    def _issue_gathers(b):
        pltpu.async_copy(sd_hbm.at[srcb[b]], sdsrc[b], gsem[b])
        pltpu.async_copy(sd_hbm.at[dstb[b]], sddst[b], gsem[b])
        pltpu.async_copy(h_hbm.at[c].at[srcb[b]], hrows[b], gsem[b])

    def _drain_gathers(b):
        pltpu.make_async_copy(sd_hbm.at[srcb[b]], sdsrc[b], gsem[b]).wait()
        pltpu.make_async_copy(sd_hbm.at[dstb[b]], sddst[b], gsem[b]).wait()
        pltpu.make_async_copy(h_hbm.at[c].at[srcb[b]], hrows[b], gsem[b]).wait()

    def _drain_scatter(b):
        pltpu.make_async_copy(scaled[b], acc.at[dsts[b]], ssem[b]).wait()

    def _compute(b):
        for jj in range(EK // 16):
            rows = lane + jj * 16
            for dd in range(ND):
                a = (plsc.load_gather(sdsrc[b], [rows, jnp.full((16,), dd, jnp.int32)])
                     + plsc.load_gather(sddst[b], [rows, jnp.full((16,), dd + 3, jnp.int32)]))
                eb[dd, pl.ds(jj * 16, 16)] = jnp.exp(jnp.maximum(a, 0.2 * a))

        @plsc.parallel_loop(0, EK)
        def _scale(j):
            jsel = jnp.full((16,), j, jnp.int32)
            ev = [plsc.load_gather(eb, [jnp.full((16,), dd, jnp.int32), jsel])
                  for dd in range(ND)]
            for q in range(FH // 16):
                dq = (c * FH + q * 16) // HID
                evq = jnp.where(dq == 0, ev[0],
                                jnp.where(dq == 1, ev[1], ev[2]))
                scaled[b][j, pl.ds(q * 16, 16)] = hrows[b][j, pl.ds(q * 16, 16)] * evq
            denv = jnp.where(lane == 0, ev[0],
                             jnp.where(lane == 1, ev[1],
                                       jnp.where(lane == 2, ev[2], 0.0)))
            scaled[b][j, pl.ds(FH, 16)] = denv

    _issue_idx(0, 0)
    _issue_idx(1, 1)
    _drain_idx(0, 0)
    _issue_gathers(0)

    def _iter(ch, b, b2):
        _drain_gathers(b)

        @pl.when(ch >= 2)
        def _():
            _drain_scatter(b)
        for jj in range(EK // 16):
            dsts[b][pl.ds(jj * 16, 16)] = dstb[b][pl.ds(jj * 16, 16)]

        @pl.when(ch + 1 < NCH)
        def _():
            _drain_idx(ch + 1, b2)
            _issue_gathers(b2)

        @pl.when(ch + 2 < NCH)
        def _():
            _issue_idx(ch + 2, b)

        _compute(b)
        pltpu.async_copy(scaled[b], acc.at[dsts[b]], ssem[b], add=True)

    def _pair(g, carry):
        _iter(2 * g, 0, 1)
        _iter(2 * g + 1, 1, 0)
        return carry
    lax.fori_loop(0, NCH // 2, _pair, 0)
    _drain_scatter(0)
    _drain_scatter(1)

    plsc.subcore_barrier()
    @pl.when(sid < 10)
    def _():
        pltpu.sync_copy(acc.at[pl.ds(sid * ZROWS, ZROWS)],
                        out_hbm.at[c].at[pl.ds(sid * ZROWS, ZROWS)])


def edge_conv(hsplit, sd, src, dst):
    mesh = plsc.VectorSubcoreMesh(core_axis_name="c", subcore_axis_name="s")
    f = pl.kernel(
        _edge_body,
        out_type=jax.ShapeDtypeStruct((2, N, OUTW), jnp.float32),
        mesh=mesh,
        compiler_params=pltpu.CompilerParams(use_tc_tiling_on_sc=False,
                                             needs_layout_passes=False),
        scratch_types=[
            (pltpu.VMEM((EK,), jnp.int32),) * 2,
            (pltpu.VMEM((EK,), jnp.int32),) * 2,
            (pltpu.VMEM((EK,), jnp.int32),) * 2,
            (pltpu.VMEM((EK, 16), jnp.float32),) * 2,
            (pltpu.VMEM((EK, 16), jnp.float32),) * 2,
            (pltpu.VMEM((EK, FH), jnp.float32),) * 2,
            (pltpu.VMEM((EK, OUTW), jnp.float32),) * 2,
            pltpu.VMEM((ND, EK), jnp.float32),
            pltpu.VMEM_SHARED((N, OUTW), jnp.float32),
            (pltpu.SemaphoreType.DMA,) * 2,
            (pltpu.SemaphoreType.DMA,) * 2,
            (pltpu.SemaphoreType.DMA,) * 2,
        ],
    )
    return f(hsplit, sd, src, dst)


def kernel(x, edge_index, disease, W1, as1, ad1, b1, W2, as2, ad2, b2, Ws, bs,
           Wsig1, bsig1, Wsig2, bsig2, Wrole1, brole1, Wrole2, brole2,
           Wab1, bab1, Wab2, bab2):
    src = edge_index[0]
    dst = edge_index[1]

    eye = jnp.eye(ND, dtype=jnp.float32)
    w1cat = jnp.transpose(W1, (1, 0, 2)).reshape(D_IN, F)
    def asd_pack(a_s, a_d):
        m = jnp.concatenate([
            jnp.einsum('dh,dc->dhc', a_s, eye).reshape(F, ND),
            jnp.einsum('dh,dc->dhc', a_d, eye).reshape(F, ND),
            jnp.zeros((F, 10), jnp.float32)], axis=1)
        return m[:FH], m[FH:]
    asd1lo, asd1hi = asd_pack(as1, ad1)
    asd2lo, asd2hi = asd_pack(as2, ad2)
    w2bd = jnp.einsum('dij,dc->dicj', W2, eye).reshape(F, F)
    expand = jnp.repeat(eye, HID, axis=1)
    explo, exphi = expand[:, :FH], expand[:, FH:]
    b1f = b1.reshape(1, F)
    b2f = b2.reshape(1, F)
    wh1 = jnp.concatenate([Wsig1[disease], Wrole1[disease], Wab1[disease]],
                          axis=1)
    bh1 = jnp.concatenate([bsig1[disease], brole1[disease], bab1[disease]],
                          axis=0).reshape(1, F)
    z64 = jnp.zeros((HID, 1), jnp.float32)
    wh2 = jnp.concatenate([
        jnp.concatenate([Wsig2[disease], z64, z64, z64], axis=1),
        jnp.concatenate([z64, Wrole2[disease], z64], axis=1),
        jnp.concatenate([z64, z64, z64, Wab2[disease]], axis=1),
    ], axis=0)
    wh2 = jnp.concatenate([wh2, jnp.zeros((F, 4), jnp.float32)], axis=1)
    bh2 = jnp.concatenate([bsig2[disease], brole2[disease], bab2[disease],
                           jnp.zeros((4,), jnp.float32)]).reshape(1, 8)

    h1, sd1 = dense1(x, w1cat[:, :FH], w1cat[:, FH:], asd1lo, asd1hi)
    o1 = edge_conv(h1, sd1, src, dst)
    h2, sd2 = dense2(o1[0], o1[1], explo, exphi, b1f[:, :FH], b1f[:, FH:],
                     w2bd[:FH, :FH], w2bd[FH:, :FH], w2bd[:FH, FH:],
                     w2bd[FH:, FH:], asd2lo, asd2hi)
    o2 = edge_conv(h2, sd2, src, dst)
    y = heads(o2[0], o2[1], explo, exphi, b2f[:, :FH], b2f[:, FH:],
              Ws[:FH], Ws[FH:], bs.reshape(1, SHARED), wh1, bh1, wh2, bh2)
    return (y[:, 0:1], y[:, 1:3], y[:, 3:4])


# speedup vs baseline: 66.0217x; 1.0066x over previous
import jax
import jax.numpy as jnp
from jax import lax
from jax.experimental import pallas as pl
from jax.experimental.pallas import tpu as pltpu
from jax.experimental.pallas import tpu_sc as plsc

N = 10000
E = 320000
D_IN = 128
HID = 64
ND = 3
SHARED = 128
F = ND * HID
FH = F // 2
ROWS = 1000


def _mm(a, b):
    return jax.lax.dot_general(a, b, (((1,), (0,)), ((), ())),
                               preferred_element_type=jnp.float32)


def _mm_hi(a, b):
    return jax.lax.dot_general(a, b, (((1,), (0,)), ((), ())),
                               preferred_element_type=jnp.float32,
                               precision=jax.lax.Precision.HIGHEST)


def _k_dense1(x_ref, wlo_ref, whi_ref, alo_ref, ahi_ref, h_ref, sd_ref):
    x = x_ref[...]
    hlo = _mm(x, wlo_ref[...])
    hhi = _mm(x, whi_ref[...])
    h_ref[0] = hlo
    h_ref[1] = hhi
    sd_ref[...] = _mm_hi(hlo, alo_ref[...]) + _mm_hi(hhi, ahi_ref[...])


def dense1(x, wlo, whi, asdlo, asdhi):
    return pl.pallas_call(
        _k_dense1,
        grid=(N // ROWS,),
        in_specs=[
            pl.BlockSpec((ROWS, D_IN), lambda i: (i, 0)),
            pl.BlockSpec((D_IN, FH), lambda i: (0, 0)),
            pl.BlockSpec((D_IN, FH), lambda i: (0, 0)),
            pl.BlockSpec((FH, 16), lambda i: (0, 0)),
            pl.BlockSpec((FH, 16), lambda i: (0, 0)),
        ],
        out_specs=[
            pl.BlockSpec((2, ROWS, FH), lambda i: (0, i, 0)),
            pl.BlockSpec((ROWS, 16), lambda i: (i, 0)),
        ],
        out_shape=[
            jax.ShapeDtypeStruct((2, N, FH), jnp.float32),
            jax.ShapeDtypeStruct((N, 16), jnp.float32),
        ],
    )(x, wlo, whi, asdlo, asdhi)


def _norm_act(out_ref, exp_ref, b_ref, relu):
    num = out_ref[...][:, :FH]
    den = out_ref[...][:, FH:FH + 3]
    den_big = _mm_hi(den, exp_ref[...])
    a = num / (den_big + 1e-16) + b_ref[...]
    return jnp.maximum(a, 0.0) if relu else a


def _k_dense2(outl_ref, outh_ref, explo_ref, exphi_ref, blo_ref, bhi_ref,
              wll_ref, whl_ref, wlh_ref, whh_ref, alo_ref, ahi_ref,
              h_ref, sd_ref):
    alo = _norm_act(outl_ref, explo_ref, blo_ref, True)
    ahi = _norm_act(outh_ref, exphi_ref, bhi_ref, True)
    hlo = _mm(alo, wll_ref[...]) + _mm(ahi, whl_ref[...])
    hhi = _mm(alo, wlh_ref[...]) + _mm(ahi, whh_ref[...])
    h_ref[0] = hlo
    h_ref[1] = hhi
    sd_ref[...] = _mm_hi(hlo, alo_ref[...]) + _mm_hi(hhi, ahi_ref[...])


def dense2(outl, outh, explo, exphi, blo, bhi, wll, whl, wlh, whh,
           asdlo, asdhi):
    return pl.pallas_call(
        _k_dense2,
        grid=(N // ROWS,),
        in_specs=[
            pl.BlockSpec((ROWS, 112), lambda i: (i, 0)),
            pl.BlockSpec((ROWS, 112), lambda i: (i, 0)),
            pl.BlockSpec((ND, FH), lambda i: (0, 0)),
            pl.BlockSpec((ND, FH), lambda i: (0, 0)),
            pl.BlockSpec((1, FH), lambda i: (0, 0)),
            pl.BlockSpec((1, FH), lambda i: (0, 0)),
            pl.BlockSpec((FH, FH), lambda i: (0, 0)),
            pl.BlockSpec((FH, FH), lambda i: (0, 0)),
            pl.BlockSpec((FH, FH), lambda i: (0, 0)),
            pl.BlockSpec((FH, FH), lambda i: (0, 0)),
            pl.BlockSpec((FH, 16), lambda i: (0, 0)),
            pl.BlockSpec((FH, 16), lambda i: (0, 0)),
        ],
        out_specs=[
            pl.BlockSpec((2, ROWS, FH), lambda i: (0, i, 0)),
            pl.BlockSpec((ROWS, 16), lambda i: (i, 0)),
        ],
        out_shape=[
            jax.ShapeDtypeStruct((2, N, FH), jnp.float32),
            jax.ShapeDtypeStruct((N, 16), jnp.float32),
        ],
    )(outl, outh, explo, exphi, blo, bhi, wll, whl, wlh, whh, asdlo, asdhi)


def _k_heads(outl_ref, outh_ref, explo_ref, exphi_ref, blo_ref, bhi_ref,
             wslo_ref, wshi_ref, bs_ref, wh1_ref, bh1_ref, wh2_ref, bh2_ref,
             y_ref):
    alo = _norm_act(outl_ref, explo_ref, blo_ref, False)
    ahi = _norm_act(outh_ref, exphi_ref, bhi_ref, False)
    z = _mm(alo, wslo_ref[...]) + _mm(ahi, wshi_ref[...]) + bs_ref[...]
    hs = jnp.where(z > 0, z, jnp.exp(jnp.minimum(z, 0.0)) - 1.0)
    t = jnp.maximum(_mm(hs, wh1_ref[...]) + bh1_ref[...], 0.0)
    y_ref[...] = _mm(t, wh2_ref[...]) + bh2_ref[...]


def heads(outl, outh, explo, exphi, blo, bhi, wslo, wshi, bs, wh1, bh1,
          wh2, bh2):
    return pl.pallas_call(
        _k_heads,
        grid=(N // ROWS,),
        in_specs=[
            pl.BlockSpec((ROWS, 112), lambda i: (i, 0)),
            pl.BlockSpec((ROWS, 112), lambda i: (i, 0)),
            pl.BlockSpec((ND, FH), lambda i: (0, 0)),
            pl.BlockSpec((ND, FH), lambda i: (0, 0)),
            pl.BlockSpec((1, FH), lambda i: (0, 0)),
            pl.BlockSpec((1, FH), lambda i: (0, 0)),
            pl.BlockSpec((FH, SHARED), lambda i: (0, 0)),
            pl.BlockSpec((FH, SHARED), lambda i: (0, 0)),
            pl.BlockSpec((1, SHARED), lambda i: (0, 0)),
            pl.BlockSpec((SHARED, F), lambda i: (0, 0)),
            pl.BlockSpec((1, F), lambda i: (0, 0)),
            pl.BlockSpec((F, 8), lambda i: (0, 0)),
            pl.BlockSpec((1, 8), lambda i: (0, 0)),
        ],
        out_specs=pl.BlockSpec((ROWS, 8), lambda i: (i, 0)),
        out_shape=jax.ShapeDtypeStruct((N, 8), jnp.float32),
    )(outl, outh, explo, exphi, blo, bhi, wslo, wshi, bs, wh1, bh1, wh2, bh2)


EK = 80
TPC = 16
EPT = E // TPC
NCH = EPT // EK
OUTW = 112
ZROWS = 1000


def _edge_body(h_hbm, sd_hbm, src_hbm, dst_hbm, out_hbm,
               srcb, dstb, dsts, sdsrc, sddst, hrows, scaled, eb, acc,
               isem, gsem, ssem):
    c = lax.axis_index("c")
    sid = lax.axis_index("s")
    base = sid * EPT

    def _zero_row(r, carry):
        for q in range(OUTW // 16):
            scaled[0][r, pl.ds(q * 16, 16)] = jnp.zeros((16,), jnp.float32)
        return carry
    lax.fori_loop(0, EK, _zero_row, 0)

    @pl.when(sid < 10)
    def _():
        for q in range(ZROWS // EK):
            pltpu.sync_copy(scaled[0], acc.at[pl.ds(sid * ZROWS + q * EK, EK)])
        rem = ZROWS - (ZROWS // EK) * EK
        if rem:
            pltpu.sync_copy(scaled[0].at[pl.ds(0, rem)],
                            acc.at[pl.ds(sid * ZROWS + (ZROWS // EK) * EK, rem)])
    plsc.subcore_barrier()

    lane = lax.broadcasted_iota(jnp.int32, (16,), 0)

    def _issue_idx(ch, b):
        off = base + ch * EK
        pltpu.async_copy(src_hbm.at[pl.ds(off, EK)], srcb[b], isem[b])
        pltpu.async_copy(dst_hbm.at[pl.ds(off, EK)], dstb[b], isem[b])

    def _drain_idx(ch, b):
        off = base + ch * EK
        pltpu.make_async_copy(src_hbm.at[pl.ds(off, EK)], srcb[b], isem[b]).wait()
        pltpu.make_async_copy(dst_hbm.at[pl.ds(off, EK)], dstb[b], isem[b]).wait()

    def _issue_gathers(b):
        pltpu.async_copy(sd_hbm.at[srcb[b]], sdsrc[b], gsem[b])
        pltpu.async_copy(sd_hbm.at[dstb[b]], sddst[b], gsem[b])
        pltpu.async_copy(h_hbm.at[c].at[srcb[b]], hrows[b], gsem[b])

    def _drain_gathers(b):
        pltpu.make_async_copy(sd_hbm.at[srcb[b]], sdsrc[b], gsem[b]).wait()
        pltpu.make_async_copy(sd_hbm.at[dstb[b]], sddst[b], gsem[b]).wait()
        pltpu.make_async_copy(h_hbm.at[c].at[srcb[b]], hrows[b], gsem[b]).wait()

    def _drain_scatter(b):
        pltpu.make_async_copy(scaled[b], acc.at[dsts[b]], ssem[b]).wait()

    def _compute(b):
        for jj in range(EK // 16):
            rows = lane + jj * 16
            for dd in range(ND):
                a = (plsc.load_gather(sdsrc[b], [rows, jnp.full((16,), dd, jnp.int32)])
                     + plsc.load_gather(sddst[b], [rows, jnp.full((16,), dd + 3, jnp.int32)]))
                eb[dd, pl.ds(jj * 16, 16)] = jnp.exp(jnp.maximum(a, 0.2 * a))

        @plsc.parallel_loop(0, EK, unroll=4)
        def _scale(j):
            jsel = jnp.full((16,), j, jnp.int32)
            ev = [plsc.load_gather(eb, [jnp.full((16,), dd, jnp.int32), jsel])
                  for dd in range(ND)]
            for q in range(FH // 16):
                dq = (c * FH + q * 16) // HID
                evq = jnp.where(dq == 0, ev[0],
                                jnp.where(dq == 1, ev[1], ev[2]))
                scaled[b][j, pl.ds(q * 16, 16)] = hrows[b][j, pl.ds(q * 16, 16)] * evq
            denv = jnp.where(lane == 0, ev[0],
                             jnp.where(lane == 1, ev[1],
                                       jnp.where(lane == 2, ev[2], 0.0)))
            scaled[b][j, pl.ds(FH, 16)] = denv

    _issue_idx(0, 0)
    _issue_idx(1, 1)
    _drain_idx(0, 0)
    _issue_gathers(0)

    def _iter(ch, b, b2):
        _drain_gathers(b)

        @pl.when(ch >= 2)
        def _():
            _drain_scatter(b)
        for jj in range(EK // 16):
            dsts[b][pl.ds(jj * 16, 16)] = dstb[b][pl.ds(jj * 16, 16)]

        @pl.when(ch + 1 < NCH)
        def _():
            _drain_idx(ch + 1, b2)
            _issue_gathers(b2)

        @pl.when(ch + 2 < NCH)
        def _():
            _issue_idx(ch + 2, b)

        _compute(b)
        pltpu.async_copy(scaled[b], acc.at[dsts[b]], ssem[b], add=True)

    def _pair(g, carry):
        _iter(2 * g, 0, 1)
        _iter(2 * g + 1, 1, 0)
        return carry
    lax.fori_loop(0, NCH // 2, _pair, 0)
    _drain_scatter(0)
    _drain_scatter(1)

    plsc.subcore_barrier()
    @pl.when(sid < 10)
    def _():
        pltpu.sync_copy(acc.at[pl.ds(sid * ZROWS, ZROWS)],
                        out_hbm.at[c].at[pl.ds(sid * ZROWS, ZROWS)])


def edge_conv(hsplit, sd, src, dst):
    mesh = plsc.VectorSubcoreMesh(core_axis_name="c", subcore_axis_name="s")
    f = pl.kernel(
        _edge_body,
        out_type=jax.ShapeDtypeStruct((2, N, OUTW), jnp.float32),
        mesh=mesh,
        compiler_params=pltpu.CompilerParams(use_tc_tiling_on_sc=False,
                                             needs_layout_passes=False),
        scratch_types=[
            (pltpu.VMEM((EK,), jnp.int32),) * 2,
            (pltpu.VMEM((EK,), jnp.int32),) * 2,
            (pltpu.VMEM((EK,), jnp.int32),) * 2,
            (pltpu.VMEM((EK, 16), jnp.float32),) * 2,
            (pltpu.VMEM((EK, 16), jnp.float32),) * 2,
            (pltpu.VMEM((EK, FH), jnp.float32),) * 2,
            (pltpu.VMEM((EK, OUTW), jnp.float32),) * 2,
            pltpu.VMEM((ND, EK), jnp.float32),
            pltpu.VMEM_SHARED((N, OUTW), jnp.float32),
            (pltpu.SemaphoreType.DMA,) * 2,
            (pltpu.SemaphoreType.DMA,) * 2,
            (pltpu.SemaphoreType.DMA,) * 2,
        ],
    )
    return f(hsplit, sd, src, dst)


def kernel(x, edge_index, disease, W1, as1, ad1, b1, W2, as2, ad2, b2, Ws, bs,
           Wsig1, bsig1, Wsig2, bsig2, Wrole1, brole1, Wrole2, brole2,
           Wab1, bab1, Wab2, bab2):
    src = edge_index[0]
    dst = edge_index[1]

    eye = jnp.eye(ND, dtype=jnp.float32)
    w1cat = jnp.transpose(W1, (1, 0, 2)).reshape(D_IN, F)
    def asd_pack(a_s, a_d):
        m = jnp.concatenate([
            jnp.einsum('dh,dc->dhc', a_s, eye).reshape(F, ND),
            jnp.einsum('dh,dc->dhc', a_d, eye).reshape(F, ND),
            jnp.zeros((F, 10), jnp.float32)], axis=1)
        return m[:FH], m[FH:]
    asd1lo, asd1hi = asd_pack(as1, ad1)
    asd2lo, asd2hi = asd_pack(as2, ad2)
    w2bd = jnp.einsum('dij,dc->dicj', W2, eye).reshape(F, F)
    expand = jnp.repeat(eye, HID, axis=1)
    explo, exphi = expand[:, :FH], expand[:, FH:]
    b1f = b1.reshape(1, F)
    b2f = b2.reshape(1, F)
    wh1 = jnp.concatenate([Wsig1[disease], Wrole1[disease], Wab1[disease]],
                          axis=1)
    bh1 = jnp.concatenate([bsig1[disease], brole1[disease], bab1[disease]],
                          axis=0).reshape(1, F)
    z64 = jnp.zeros((HID, 1), jnp.float32)
    wh2 = jnp.concatenate([
        jnp.concatenate([Wsig2[disease], z64, z64, z64], axis=1),
        jnp.concatenate([z64, Wrole2[disease], z64], axis=1),
        jnp.concatenate([z64, z64, z64, Wab2[disease]], axis=1),
    ], axis=0)
    wh2 = jnp.concatenate([wh2, jnp.zeros((F, 4), jnp.float32)], axis=1)
    bh2 = jnp.concatenate([bsig2[disease], brole2[disease], bab2[disease],
                           jnp.zeros((4,), jnp.float32)]).reshape(1, 8)

    h1, sd1 = dense1(x, w1cat[:, :FH], w1cat[:, FH:], asd1lo, asd1hi)
    o1 = edge_conv(h1, sd1, src, dst)
    h2, sd2 = dense2(o1[0], o1[1], explo, exphi, b1f[:, :FH], b1f[:, FH:],
                     w2bd[:FH, :FH], w2bd[FH:, :FH], w2bd[:FH, FH:],
                     w2bd[FH:, FH:], asd2lo, asd2hi)
    o2 = edge_conv(h2, sd2, src, dst)
    y = heads(o2[0], o2[1], explo, exphi, b2f[:, :FH], b2f[:, FH:],
              Ws[:FH], Ws[FH:], bs.reshape(1, SHARED), wh1, bh1, wh2, bh2)
    return (y[:, 0:1], y[:, 1:3], y[:, 3:4])


# speedup vs baseline: 70.3056x; 1.0649x over previous
import jax
import jax.numpy as jnp
from jax import lax
from jax.experimental import pallas as pl
from jax.experimental.pallas import tpu as pltpu
from jax.experimental.pallas import tpu_sc as plsc

N = 10000
E = 320000
D_IN = 128
HID = 64
ND = 3
SHARED = 128
F = ND * HID
FH = F // 2
ROWS = 2000


def _mm(a, b):
    return jax.lax.dot_general(a, b, (((1,), (0,)), ((), ())),
                               preferred_element_type=jnp.float32)


def _mm_hi(a, b):
    return jax.lax.dot_general(a, b, (((1,), (0,)), ((), ())),
                               preferred_element_type=jnp.float32,
                               precision=jax.lax.Precision.HIGHEST)


def _k_dense1(x_ref, wlo_ref, whi_ref, alo_ref, ahi_ref, h_ref, sd_ref):
    x = x_ref[...]
    hlo = _mm(x, wlo_ref[...])
    hhi = _mm(x, whi_ref[...])
    h_ref[0] = hlo
    h_ref[1] = hhi
    sd_ref[...] = _mm_hi(hlo, alo_ref[...]) + _mm_hi(hhi, ahi_ref[...])


def dense1(x, wlo, whi, asdlo, asdhi):
    return pl.pallas_call(
        _k_dense1,
        grid=(N // ROWS,),
        in_specs=[
            pl.BlockSpec((ROWS, D_IN), lambda i: (i, 0)),
            pl.BlockSpec((D_IN, FH), lambda i: (0, 0)),
            pl.BlockSpec((D_IN, FH), lambda i: (0, 0)),
            pl.BlockSpec((FH, 16), lambda i: (0, 0)),
            pl.BlockSpec((FH, 16), lambda i: (0, 0)),
        ],
        out_specs=[
            pl.BlockSpec((2, ROWS, FH), lambda i: (0, i, 0)),
            pl.BlockSpec((ROWS, 16), lambda i: (i, 0)),
        ],
        out_shape=[
            jax.ShapeDtypeStruct((2, N, FH), jnp.float32),
            jax.ShapeDtypeStruct((N, 16), jnp.float32),
        ],
    )(x, wlo, whi, asdlo, asdhi)


def _norm_act(out_ref, exp_ref, b_ref, relu):
    num = out_ref[...][:, :FH]
    den = out_ref[...][:, FH:FH + 3]
    den_big = _mm_hi(den, exp_ref[...])
    a = num / (den_big + 1e-16) + b_ref[...]
    return jnp.maximum(a, 0.0) if relu else a


def _k_dense2(outl_ref, outh_ref, explo_ref, exphi_ref, blo_ref, bhi_ref,
              wll_ref, whl_ref, wlh_ref, whh_ref, alo_ref, ahi_ref,
              h_ref, sd_ref):
    alo = _norm_act(outl_ref, explo_ref, blo_ref, True)
    ahi = _norm_act(outh_ref, exphi_ref, bhi_ref, True)
    hlo = _mm(alo, wll_ref[...]) + _mm(ahi, whl_ref[...])
    hhi = _mm(alo, wlh_ref[...]) + _mm(ahi, whh_ref[...])
    h_ref[0] = hlo
    h_ref[1] = hhi
    sd_ref[...] = _mm_hi(hlo, alo_ref[...]) + _mm_hi(hhi, ahi_ref[...])


def dense2(outl, outh, explo, exphi, blo, bhi, wll, whl, wlh, whh,
           asdlo, asdhi):
    return pl.pallas_call(
        _k_dense2,
        grid=(N // ROWS,),
        in_specs=[
            pl.BlockSpec((ROWS, 112), lambda i: (i, 0)),
            pl.BlockSpec((ROWS, 112), lambda i: (i, 0)),
            pl.BlockSpec((ND, FH), lambda i: (0, 0)),
            pl.BlockSpec((ND, FH), lambda i: (0, 0)),
            pl.BlockSpec((1, FH), lambda i: (0, 0)),
            pl.BlockSpec((1, FH), lambda i: (0, 0)),
            pl.BlockSpec((FH, FH), lambda i: (0, 0)),
            pl.BlockSpec((FH, FH), lambda i: (0, 0)),
            pl.BlockSpec((FH, FH), lambda i: (0, 0)),
            pl.BlockSpec((FH, FH), lambda i: (0, 0)),
            pl.BlockSpec((FH, 16), lambda i: (0, 0)),
            pl.BlockSpec((FH, 16), lambda i: (0, 0)),
        ],
        out_specs=[
            pl.BlockSpec((2, ROWS, FH), lambda i: (0, i, 0)),
            pl.BlockSpec((ROWS, 16), lambda i: (i, 0)),
        ],
        out_shape=[
            jax.ShapeDtypeStruct((2, N, FH), jnp.float32),
            jax.ShapeDtypeStruct((N, 16), jnp.float32),
        ],
    )(outl, outh, explo, exphi, blo, bhi, wll, whl, wlh, whh, asdlo, asdhi)


def _k_heads(outl_ref, outh_ref, explo_ref, exphi_ref, blo_ref, bhi_ref,
             wslo_ref, wshi_ref, bs_ref, wh1_ref, bh1_ref, wh2_ref, bh2_ref,
             y_ref):
    alo = _norm_act(outl_ref, explo_ref, blo_ref, False)
    ahi = _norm_act(outh_ref, exphi_ref, bhi_ref, False)
    z = _mm(alo, wslo_ref[...]) + _mm(ahi, wshi_ref[...]) + bs_ref[...]
    hs = jnp.where(z > 0, z, jnp.exp(jnp.minimum(z, 0.0)) - 1.0)
    t = jnp.maximum(_mm(hs, wh1_ref[...]) + bh1_ref[...], 0.0)
    y_ref[...] = _mm(t, wh2_ref[...]) + bh2_ref[...]


def heads(outl, outh, explo, exphi, blo, bhi, wslo, wshi, bs, wh1, bh1,
          wh2, bh2):
    return pl.pallas_call(
        _k_heads,
        grid=(N // ROWS,),
        in_specs=[
            pl.BlockSpec((ROWS, 112), lambda i: (i, 0)),
            pl.BlockSpec((ROWS, 112), lambda i: (i, 0)),
            pl.BlockSpec((ND, FH), lambda i: (0, 0)),
            pl.BlockSpec((ND, FH), lambda i: (0, 0)),
            pl.BlockSpec((1, FH), lambda i: (0, 0)),
            pl.BlockSpec((1, FH), lambda i: (0, 0)),
            pl.BlockSpec((FH, SHARED), lambda i: (0, 0)),
            pl.BlockSpec((FH, SHARED), lambda i: (0, 0)),
            pl.BlockSpec((1, SHARED), lambda i: (0, 0)),
            pl.BlockSpec((SHARED, F), lambda i: (0, 0)),
            pl.BlockSpec((1, F), lambda i: (0, 0)),
            pl.BlockSpec((F, 8), lambda i: (0, 0)),
            pl.BlockSpec((1, 8), lambda i: (0, 0)),
        ],
        out_specs=pl.BlockSpec((ROWS, 8), lambda i: (i, 0)),
        out_shape=jax.ShapeDtypeStruct((N, 8), jnp.float32),
    )(outl, outh, explo, exphi, blo, bhi, wslo, wshi, bs, wh1, bh1, wh2, bh2)


EK = 80
TPC = 16
EPT = E // TPC
NCH = EPT // EK
OUTW = 112
ZROWS = 1000


def _edge_body(h_hbm, sd_hbm, src_hbm, dst_hbm, out_hbm,
               srcb, dstb, dsts, sdsrc, sddst, hrows, scaled, eb, acc,
               isem, gsem, ssem):
    c = lax.axis_index("c")
    sid = lax.axis_index("s")
    base = sid * EPT

    def _zero_row(r, carry):
        for q in range(OUTW // 16):
            scaled[0][r, pl.ds(q * 16, 16)] = jnp.zeros((16,), jnp.float32)
        return carry
    lax.fori_loop(0, EK, _zero_row, 0)

    @pl.when(sid < 10)
    def _():
        for q in range(ZROWS // EK):
            pltpu.sync_copy(scaled[0], acc.at[pl.ds(sid * ZROWS + q * EK, EK)])
        rem = ZROWS - (ZROWS // EK) * EK
        if rem:
            pltpu.sync_copy(scaled[0].at[pl.ds(0, rem)],
                            acc.at[pl.ds(sid * ZROWS + (ZROWS // EK) * EK, rem)])
    plsc.subcore_barrier()

    lane = lax.broadcasted_iota(jnp.int32, (16,), 0)

    def _issue_idx(ch, b):
        off = base + ch * EK
        pltpu.async_copy(src_hbm.at[pl.ds(off, EK)], srcb[b], isem[b])
        pltpu.async_copy(dst_hbm.at[pl.ds(off, EK)], dstb[b], isem[b])

    def _drain_idx(ch, b):
        off = base + ch * EK
        pltpu.make_async_copy(src_hbm.at[pl.ds(off, EK)], srcb[b], isem[b]).wait()
        pltpu.make_async_copy(dst_hbm.at[pl.ds(off, EK)], dstb[b], isem[b]).wait()

    def _issue_gathers(b):
        pltpu.async_copy(sd_hbm.at[srcb[b]], sdsrc[b], gsem[b])
        pltpu.async_copy(sd_hbm.at[dstb[b]], sddst[b], gsem[b])
        pltpu.async_copy(h_hbm.at[c].at[srcb[b]], hrows[b], gsem[b])

    def _drain_gathers(b):
        pltpu.make_async_copy(sd_hbm.at[srcb[b]], sdsrc[b], gsem[b]).wait()
        pltpu.make_async_copy(sd_hbm.at[dstb[b]], sddst[b], gsem[b]).wait()
        pltpu.make_async_copy(h_hbm.at[c].at[srcb[b]], hrows[b], gsem[b]).wait()

    def _drain_scatter(b):
        pltpu.make_async_copy(scaled[b], acc.at[dsts[b]], ssem[b]).wait()

    def _compute(b):
        for jj in range(EK // 16):
            rows = lane + jj * 16
            for dd in range(ND):
                a = (plsc.load_gather(sdsrc[b], [rows, jnp.full((16,), dd, jnp.int32)])
                     + plsc.load_gather(sddst[b], [rows, jnp.full((16,), dd + 3, jnp.int32)]))
                eb[dd, pl.ds(jj * 16, 16)] = jnp.exp(jnp.maximum(a, 0.2 * a))

        @plsc.parallel_loop(0, EK, unroll=4)
        def _scale(j):
            jsel = jnp.full((16,), j, jnp.int32)
            ev = [plsc.load_gather(eb, [jnp.full((16,), dd, jnp.int32), jsel])
                  for dd in range(ND)]
            for q in range(FH // 16):
                dq = (c * FH + q * 16) // HID
                evq = jnp.where(dq == 0, ev[0],
                                jnp.where(dq == 1, ev[1], ev[2]))
                scaled[b][j, pl.ds(q * 16, 16)] = hrows[b][j, pl.ds(q * 16, 16)] * evq
            denv = jnp.where(lane == 0, ev[0],
                             jnp.where(lane == 1, ev[1],
                                       jnp.where(lane == 2, ev[2], 0.0)))
            scaled[b][j, pl.ds(FH, 16)] = denv

    _issue_idx(0, 0)
    _issue_idx(1, 1)
    _drain_idx(0, 0)
    _issue_gathers(0)

    def _iter(ch, b, b2):
        _drain_gathers(b)

        @pl.when(ch >= 2)
        def _():
            _drain_scatter(b)
        for jj in range(EK // 16):
            dsts[b][pl.ds(jj * 16, 16)] = dstb[b][pl.ds(jj * 16, 16)]

        @pl.when(ch + 1 < NCH)
        def _():
            _drain_idx(ch + 1, b2)
            _issue_gathers(b2)

        @pl.when(ch + 2 < NCH)
        def _():
            _issue_idx(ch + 2, b)

        _compute(b)
        pltpu.async_copy(scaled[b], acc.at[dsts[b]], ssem[b], add=True)

    def _pair(g, carry):
        _iter(2 * g, 0, 1)
        _iter(2 * g + 1, 1, 0)
        return carry
    lax.fori_loop(0, NCH // 2, _pair, 0)
    _drain_scatter(0)
    _drain_scatter(1)

    plsc.subcore_barrier()
    @pl.when(sid < 10)
    def _():
        pltpu.sync_copy(acc.at[pl.ds(sid * ZROWS, ZROWS)],
                        out_hbm.at[c].at[pl.ds(sid * ZROWS, ZROWS)])


def edge_conv(hsplit, sd, src, dst):
    mesh = plsc.VectorSubcoreMesh(core_axis_name="c", subcore_axis_name="s")
    f = pl.kernel(
        _edge_body,
        out_type=jax.ShapeDtypeStruct((2, N, OUTW), jnp.float32),
        mesh=mesh,
        compiler_params=pltpu.CompilerParams(use_tc_tiling_on_sc=False,
                                             needs_layout_passes=False),
        scratch_types=[
            (pltpu.VMEM((EK,), jnp.int32),) * 2,
            (pltpu.VMEM((EK,), jnp.int32),) * 2,
            (pltpu.VMEM((EK,), jnp.int32),) * 2,
            (pltpu.VMEM((EK, 16), jnp.float32),) * 2,
            (pltpu.VMEM((EK, 16), jnp.float32),) * 2,
            (pltpu.VMEM((EK, FH), jnp.float32),) * 2,
            (pltpu.VMEM((EK, OUTW), jnp.float32),) * 2,
            pltpu.VMEM((ND, EK), jnp.float32),
            pltpu.VMEM_SHARED((N, OUTW), jnp.float32),
            (pltpu.SemaphoreType.DMA,) * 2,
            (pltpu.SemaphoreType.DMA,) * 2,
            (pltpu.SemaphoreType.DMA,) * 2,
        ],
    )
    return f(hsplit, sd, src, dst)


def kernel(x, edge_index, disease, W1, as1, ad1, b1, W2, as2, ad2, b2, Ws, bs,
           Wsig1, bsig1, Wsig2, bsig2, Wrole1, brole1, Wrole2, brole2,
           Wab1, bab1, Wab2, bab2):
    src = edge_index[0]
    dst = edge_index[1]

    eye = jnp.eye(ND, dtype=jnp.float32)
    w1cat = jnp.transpose(W1, (1, 0, 2)).reshape(D_IN, F)
    def asd_pack(a_s, a_d):
        m = jnp.concatenate([
            jnp.einsum('dh,dc->dhc', a_s, eye).reshape(F, ND),
            jnp.einsum('dh,dc->dhc', a_d, eye).reshape(F, ND),
            jnp.zeros((F, 10), jnp.float32)], axis=1)
        return m[:FH], m[FH:]
    asd1lo, asd1hi = asd_pack(as1, ad1)
    asd2lo, asd2hi = asd_pack(as2, ad2)
    w2bd = jnp.einsum('dij,dc->dicj', W2, eye).reshape(F, F)
    expand = jnp.repeat(eye, HID, axis=1)
    explo, exphi = expand[:, :FH], expand[:, FH:]
    b1f = b1.reshape(1, F)
    b2f = b2.reshape(1, F)
    wh1 = jnp.concatenate([Wsig1[disease], Wrole1[disease], Wab1[disease]],
                          axis=1)
    bh1 = jnp.concatenate([bsig1[disease], brole1[disease], bab1[disease]],
                          axis=0).reshape(1, F)
    z64 = jnp.zeros((HID, 1), jnp.float32)
    wh2 = jnp.concatenate([
        jnp.concatenate([Wsig2[disease], z64, z64, z64], axis=1),
        jnp.concatenate([z64, Wrole2[disease], z64], axis=1),
        jnp.concatenate([z64, z64, z64, Wab2[disease]], axis=1),
    ], axis=0)
    wh2 = jnp.concatenate([wh2, jnp.zeros((F, 4), jnp.float32)], axis=1)
    bh2 = jnp.concatenate([bsig2[disease], brole2[disease], bab2[disease],
                           jnp.zeros((4,), jnp.float32)]).reshape(1, 8)

    h1, sd1 = dense1(x, w1cat[:, :FH], w1cat[:, FH:], asd1lo, asd1hi)
    o1 = edge_conv(h1, sd1, src, dst)
    h2, sd2 = dense2(o1[0], o1[1], explo, exphi, b1f[:, :FH], b1f[:, FH:],
                     w2bd[:FH, :FH], w2bd[FH:, :FH], w2bd[:FH, FH:],
                     w2bd[FH:, FH:], asd2lo, asd2hi)
    o2 = edge_conv(h2, sd2, src, dst)
    y = heads(o2[0], o2[1], explo, exphi, b2f[:, :FH], b2f[:, FH:],
              Ws[:FH], Ws[FH:], bs.reshape(1, SHARED), wh1, bh1, wh2, bh2)
    return (y[:, 0:1], y[:, 1:3], y[:, 3:4])
